# baseline (speedup 1.0000x reference)
import functools

import jax
import jax.numpy as jnp
from jax import lax
from jax.experimental import pallas as pl
from jax.experimental.pallas import tpu as pltpu
from jax.experimental.pallas import tpu_sc as plsc

_TOPK = 1024
_ROWS = 64
_COLS = 32768
_NW = 32
_RPW = _ROWS // _NW
_NV = _COLS // 16


def _suffix_scan(hist, k):
    def cond(st):
        return st[1] < k

    def body(st):
        b, acc, _ = st
        cvec = hist[pl.ds(b * 16, 16)]
        s = jnp.sum(cvec)
        return (b - 1, acc + s, s)

    b, acc, cnt = lax.while_loop(
        cond, body, (jnp.int32(255), jnp.int32(0), jnp.int32(0))
    )
    bucket = b + 1
    acc_above = acc - cnt
    return bucket, k - acc_above


def _sc_body(x_hbm, out_hbm, xv, kv, hist):
    c = lax.axis_index("c")
    s = lax.axis_index("s")
    wid = s * 2 + c
    lane = lax.iota(jnp.int32, 16)
    ones = jnp.ones((16,), jnp.int32)
    zeros16 = jnp.zeros((16,), jnp.int32)

    def zero_hist(i, carry):
        hist[pl.ds(i * 16, 16)] = zeros16
        return carry

    for rr in range(_RPW):
        base = (wid * _RPW + rr) * _COLS
        pltpu.sync_copy(x_hbm.at[pl.ds(base, _COLS)], xv)

        lax.fori_loop(0, 256, zero_hist, 0)

        def p1(i, carry):
            xc = xv[pl.ds(i * 16, 16)]
            u = lax.bitcast_convert_type(xc, jnp.uint32)
            sign = u >> jnp.uint32(31)
            key = u ^ (jnp.uint32(0x80000000) + sign * jnp.uint32(0x7FFFFFFF))
            kv[pl.ds(i * 16, 16)] = key
            b = (key >> jnp.uint32(24)).astype(jnp.int32)
            plsc.addupdate_scatter(hist, [b * 16 + lane], ones)
            return carry

        lax.fori_loop(0, _NV, p1, 0)
        bkt, k = _suffix_scan(hist, jnp.int32(_TOPK))
        prefix = bkt.astype(jnp.uint32)

        for p in range(1, 4):
            shift = 24 - 8 * p
            lax.fori_loop(0, 256, zero_hist, 0)

            def pp(i, carry, shift=shift, prefix=prefix):
                key = kv[pl.ds(i * 16, 16)]
                elig = (key >> jnp.uint32(shift + 8)) == prefix
                b = ((key >> jnp.uint32(shift)) & jnp.uint32(0xFF)).astype(
                    jnp.int32
                )
                plsc.addupdate_scatter(hist, [b * 16 + lane], ones, mask=elig)
                return carry

            lax.fori_loop(0, _NV, pp, 0)
            bkt, k = _suffix_scan(hist, k)
            prefix = (prefix << jnp.uint32(8)) | bkt.astype(jnp.uint32)

        thresh = prefix

        def po(i, carry):
            key = kv[pl.ds(i * 16, 16)]
            xc = xv[pl.ds(i * 16, 16)]
            xv[pl.ds(i * 16, 16)] = jnp.where(
                key >= thresh, xc, jnp.float32(0.0)
            )
            return carry

        lax.fori_loop(0, _NV, po, 0)
        pltpu.sync_copy(xv, out_hbm.at[pl.ds(base, _COLS)])


_sc_kernel = functools.partial(
    pl.kernel,
    out_type=jax.ShapeDtypeStruct((_ROWS * _COLS,), jnp.float32),
    mesh=plsc.VectorSubcoreMesh(core_axis_name="c", subcore_axis_name="s"),
    scratch_types=[
        pltpu.VMEM((_COLS,), jnp.float32),
        pltpu.VMEM((_COLS,), jnp.uint32),
        pltpu.VMEM((16 * 256,), jnp.int32),
    ],
    compiler_params=pltpu.CompilerParams(needs_layout_passes=False),
)(_sc_body)


def kernel(x):
    return _sc_kernel(x.reshape(-1)).reshape(_ROWS, _COLS)


# speedup vs baseline: 2.6197x; 2.6197x over previous
import functools

import jax
import jax.numpy as jnp
from jax import lax
from jax.experimental import pallas as pl
from jax.experimental.pallas import tpu as pltpu
from jax.experimental.pallas import tpu_sc as plsc

_TOPK = 1024
_ROWS = 64
_COLS = 32768
_NW = 32
_RPW = _ROWS // _NW
_NV = _COLS // 16
_UNROLL = 8


def _suffix_scan(hist, k):
    def cond(st):
        return st[1] < k

    def body(st):
        b, acc, _ = st
        cvec = hist[pl.ds(b * 16, 16)]
        s = jnp.sum(cvec)
        return (b - 1, acc + s, s)

    b, acc, cnt = lax.while_loop(
        cond, body, (jnp.int32(255), jnp.int32(0), jnp.int32(0))
    )
    bucket = b + 1
    acc_above = acc - cnt
    return bucket, k - acc_above


def _sc_body(x_hbm, out_hbm, xv, kv, hist):
    c = lax.axis_index("c")
    s = lax.axis_index("s")
    wid = s * 2 + c
    lane = lax.iota(jnp.int32, 16)
    ones = jnp.ones((16,), jnp.int32)
    zeros16 = jnp.zeros((16,), jnp.int32)

    def zero_hist():
        @plsc.parallel_loop(0, 256, unroll=_UNROLL)
        def _(i):
            hist[pl.ds(i * 16, 16)] = zeros16

    for rr in range(_RPW):
        base = (wid * _RPW + rr) * _COLS
        pltpu.sync_copy(x_hbm.at[pl.ds(base, _COLS)], xv)

        zero_hist()

        @plsc.parallel_loop(0, _NV, unroll=_UNROLL)
        def _(i):
            xc = xv[pl.ds(i * 16, 16)]
            u = lax.bitcast_convert_type(xc, jnp.uint32)
            sign = u >> jnp.uint32(31)
            key = u ^ (jnp.uint32(0x80000000) + sign * jnp.uint32(0x7FFFFFFF))
            kv[pl.ds(i * 16, 16)] = key
            slot = ((key >> jnp.uint32(20)) & jnp.uint32(0xFF0)).astype(
                jnp.int32
            )
            plsc.addupdate_scatter(hist, [slot + lane], ones)

        bkt, k = _suffix_scan(hist, jnp.int32(_TOPK))
        prefix = bkt.astype(jnp.uint32)

        for p in range(1, 4):
            shift = 24 - 8 * p
            zero_hist()

            @plsc.parallel_loop(0, _NV, unroll=_UNROLL)
            def _(i, shift=shift, prefix=prefix):
                key = kv[pl.ds(i * 16, 16)]
                elig = (key >> jnp.uint32(shift + 8)) == prefix
                if shift >= 4:
                    slot = (key >> jnp.uint32(shift - 4)) & jnp.uint32(0xFF0)
                else:
                    slot = (key << jnp.uint32(4)) & jnp.uint32(0xFF0)
                plsc.addupdate_scatter(
                    hist, [slot.astype(jnp.int32) + lane], ones, mask=elig
                )

            bkt, k = _suffix_scan(hist, k)
            prefix = (prefix << jnp.uint32(8)) | bkt.astype(jnp.uint32)

        thresh = prefix

        @plsc.parallel_loop(0, _NV, unroll=_UNROLL)
        def _(i):
            key = kv[pl.ds(i * 16, 16)]
            xc = xv[pl.ds(i * 16, 16)]
            xv[pl.ds(i * 16, 16)] = jnp.where(
                key >= thresh, xc, jnp.float32(0.0)
            )

        pltpu.sync_copy(xv, out_hbm.at[pl.ds(base, _COLS)])


_sc_kernel = functools.partial(
    pl.kernel,
    out_type=jax.ShapeDtypeStruct((_ROWS * _COLS,), jnp.float32),
    mesh=plsc.VectorSubcoreMesh(core_axis_name="c", subcore_axis_name="s"),
    scratch_types=[
        pltpu.VMEM((_COLS,), jnp.float32),
        pltpu.VMEM((_COLS,), jnp.uint32),
        pltpu.VMEM((16 * 256,), jnp.int32),
    ],
    compiler_params=pltpu.CompilerParams(needs_layout_passes=False),
)(_sc_body)


def kernel(x):
    return _sc_kernel(x.reshape(-1)).reshape(_ROWS, _COLS)
